# baseline (speedup 1.0000x reference)
import functools

import jax
import jax.numpy as jnp
from jax.experimental import pallas as pl
from jax.experimental.pallas import tpu as pltpu


def _bilstm_body(T, B, I, H, O,
                 x_ref,
                 wi_f_ref,
                 wi_b_ref,
                 wh_f_ref,
                 wh_b_ref,
                 b_f_ref,
                 b_b_ref,
                 wl_f_ref,
                 wl_b_ref,
                 bl_ref,
                 o_ref,
                 ginb_scr,
                 hf_scr,
                 h_ref,
                 c_ref):
    f32 = jnp.float32
    bf16 = jnp.bfloat16
    k = pl.program_id(0)
    Th = T // 4

    col = jax.lax.broadcasted_iota(jnp.int32, (1, 4 * H), 1)
    gscale = jnp.where((col >= 2 * H) & (col < 3 * H), 1.0, 0.5).astype(f32)

    def gate_act(gates):
        th = jnp.tanh(gates * gscale)
        i_g = th[:, 0 * H:1 * H] * 0.5 + 0.5
        f_g = th[:, 1 * H:2 * H] * 0.5 + 0.5
        g_g = th[:, 2 * H:3 * H]
        o_g = th[:, 3 * H:4 * H] * 0.5 + 0.5
        return i_g, f_g, g_g, o_g

    @pl.when(k == 0)
    def _init():
        h_ref[...] = jnp.zeros((B, H), f32)
        c_ref[...] = jnp.zeros((B, H), f32)

    wi_f = wi_f_ref[...].astype(bf16)
    wi_b = wi_b_ref[...].astype(bf16)
    wh_f = wh_f_ref[...]

    h = h_ref[...]
    c = c_ref[...]
    for tt in range(Th):
        xs = x_ref[:, tt * I:(tt + 1) * I].astype(bf16)
        g_f = jnp.dot(xs, wi_f, preferred_element_type=f32) + b_f_ref[...]
        g_b = jnp.dot(xs, wi_b, preferred_element_type=f32) + b_b_ref[...]
        row = k * Th * B + tt * B
        ginb_scr[pl.ds(row, B), :] = g_b

        gates = g_f + jnp.dot(h, wh_f, preferred_element_type=f32)
        i_g, f_g, g_g, o_g = gate_act(gates)
        c = f_g * c + i_g * g_g
        h = o_g * jnp.tanh(c)
        hf_scr[pl.ds(row, B), :] = h
    h_ref[...] = h
    c_ref[...] = c

    @pl.when(k == 3)
    def _finale():
        wh_b = wh_b_ref[...]
        wl_f = wl_f_ref[...]
        wl_b = wl_b_ref[...]
        bl = bl_ref[...]
        hb = jnp.zeros((B, H), f32)
        cb = jnp.zeros((B, H), f32)
        for t in range(T - 1, -1, -1):
            g = ginb_scr[t * B:(t + 1) * B, :]
            gates_b = g + jnp.dot(hb, wh_b, preferred_element_type=f32)
            ib, fb, gb, ob = gate_act(gates_b)
            cb = fb * cb + ib * gb
            hb = ob * jnp.tanh(cb)
            hf = hf_scr[t * B:(t + 1) * B, :]
            o_ref[:, t * O:(t + 1) * O] = (
                jnp.dot(hf, wl_f, preferred_element_type=f32)
                + jnp.dot(hb, wl_b, preferred_element_type=f32) + bl)


@jax.jit
def kernel(x, wi_f, wh_f, b_f, wi_b, wh_b, b_b, wl_f, wl_b, b_lin):
    B, T, I = x.shape
    H = wh_f.shape[0]
    O = b_lin.shape[-1]
    f32 = jnp.float32

    x2 = x.reshape(B, T * I)

    def whole(shape):
        return pl.BlockSpec(shape, lambda k, _n=len(shape): (0,) * _n)

    out = pl.pallas_call(
        functools.partial(_bilstm_body, T, B, I, H, O),
        out_shape=jax.ShapeDtypeStruct((B, T * O), f32),
        grid_spec=pltpu.PrefetchScalarGridSpec(
            num_scalar_prefetch=0,
            grid=(4,),
            in_specs=[
                pl.BlockSpec((B, (T // 4) * I), lambda k: (0, k)),
                whole((I, 4 * H)),
                whole((I, 4 * H)),
                whole((H, 4 * H)),
                whole((H, 4 * H)),
                whole((1, 4 * H)),
                whole((1, 4 * H)),
                whole((H, O)),
                whole((H, O)),
                whole((1, O)),
            ],
            out_specs=whole((B, T * O)),
            scratch_shapes=[
                pltpu.VMEM((T * B, 4 * H), f32),
                pltpu.VMEM((T * B, H), f32),
                pltpu.VMEM((B, H), f32),
                pltpu.VMEM((B, H), f32),
            ],
        ),
        compiler_params=pltpu.CompilerParams(
            dimension_semantics=("arbitrary",)),
    )(x2, wi_f, wi_b, wh_f, wh_b, b_f, b_b, wl_f, wl_b, b_lin)

    return out.reshape(B, T, O)


# speedup vs baseline: 1.0076x; 1.0076x over previous
import functools

import jax
import jax.numpy as jnp
from jax.experimental import pallas as pl
from jax.experimental.pallas import tpu as pltpu


def _bilstm_body(T, B, I, H, O,
                 x_ref,
                 wi_f_ref,
                 wi_b_ref,
                 wh_f_ref,
                 wh_b_ref,
                 b_f_ref,
                 b_b_ref,
                 wl_f_ref,
                 wl_b_ref,
                 bl_ref,
                 o_ref,
                 ginb_scr,
                 hf_scr,
                 h_ref,
                 c_ref):
    f32 = jnp.float32
    bf16 = jnp.bfloat16
    k = pl.program_id(0)
    Th = T // 2

    col = jax.lax.broadcasted_iota(jnp.int32, (1, 4 * H), 1)
    gscale = jnp.where((col >= 2 * H) & (col < 3 * H), 1.0, 0.5).astype(f32)

    def gate_act(gates):
        th = jnp.tanh(gates * gscale)
        i_g = th[:, 0 * H:1 * H] * 0.5 + 0.5
        f_g = th[:, 1 * H:2 * H] * 0.5 + 0.5
        g_g = th[:, 2 * H:3 * H]
        o_g = th[:, 3 * H:4 * H] * 0.5 + 0.5
        return i_g, f_g, g_g, o_g

    @pl.when(k == 0)
    def _init():
        h_ref[...] = jnp.zeros((B, H), f32)
        c_ref[...] = jnp.zeros((B, H), f32)

    wi_f = wi_f_ref[...].astype(bf16)
    wi_b = wi_b_ref[...].astype(bf16)
    wh_f = wh_f_ref[...]

    h = h_ref[...]
    c = c_ref[...]
    for tt in range(Th):
        xs = x_ref[:, tt * I:(tt + 1) * I].astype(bf16)
        g_f = jnp.dot(xs, wi_f, preferred_element_type=f32) + b_f_ref[...]
        g_b = jnp.dot(xs, wi_b, preferred_element_type=f32) + b_b_ref[...]
        row = k * Th * B + tt * B
        ginb_scr[pl.ds(row, B), :] = g_b

        gates = g_f + jnp.dot(h, wh_f, preferred_element_type=f32)
        i_g, f_g, g_g, o_g = gate_act(gates)
        c = f_g * c + i_g * g_g
        h = o_g * jnp.tanh(c)
        hf_scr[pl.ds(row, B), :] = h
    h_ref[...] = h
    c_ref[...] = c

    @pl.when(k == 1)
    def _finale():
        wh_b = wh_b_ref[...]
        wl_f = wl_f_ref[...]
        wl_b = wl_b_ref[...]
        bl = bl_ref[...]
        hb = jnp.zeros((B, H), f32)
        cb = jnp.zeros((B, H), f32)
        for t in range(T - 1, -1, -1):
            g = ginb_scr[t * B:(t + 1) * B, :]
            gates_b = g + jnp.dot(hb, wh_b, preferred_element_type=f32)
            ib, fb, gb, ob = gate_act(gates_b)
            cb = fb * cb + ib * gb
            hb = ob * jnp.tanh(cb)
            hf = hf_scr[t * B:(t + 1) * B, :]
            o_ref[:, t * O:(t + 1) * O] = (
                jnp.dot(hf, wl_f, preferred_element_type=f32)
                + jnp.dot(hb, wl_b, preferred_element_type=f32) + bl)


@jax.jit
def kernel(x, wi_f, wh_f, b_f, wi_b, wh_b, b_b, wl_f, wl_b, b_lin):
    B, T, I = x.shape
    H = wh_f.shape[0]
    O = b_lin.shape[-1]
    f32 = jnp.float32

    x2 = x.reshape(B, T * I)

    def whole(shape):
        return pl.BlockSpec(shape, lambda k, _n=len(shape): (0,) * _n)

    out = pl.pallas_call(
        functools.partial(_bilstm_body, T, B, I, H, O),
        out_shape=jax.ShapeDtypeStruct((B, T * O), f32),
        grid_spec=pltpu.PrefetchScalarGridSpec(
            num_scalar_prefetch=0,
            grid=(2,),
            in_specs=[
                pl.BlockSpec((B, (T // 2) * I), lambda k: (0, k)),
                whole((I, 4 * H)),
                whole((I, 4 * H)),
                whole((H, 4 * H)),
                whole((H, 4 * H)),
                whole((1, 4 * H)),
                whole((1, 4 * H)),
                whole((H, O)),
                whole((H, O)),
                whole((1, O)),
            ],
            out_specs=whole((B, T * O)),
            scratch_shapes=[
                pltpu.VMEM((T * B, 4 * H), f32),
                pltpu.VMEM((T * B, H), f32),
                pltpu.VMEM((B, H), f32),
                pltpu.VMEM((B, H), f32),
            ],
        ),
        compiler_params=pltpu.CompilerParams(
            dimension_semantics=("arbitrary",)),
    )(x2, wi_f, wi_b, wh_f, wh_b, b_f, b_b, wl_f, wl_b, b_lin)

    return out.reshape(B, T, O)
